# baseline (speedup 1.0000x reference)
import functools

import jax
import jax.numpy as jnp
from jax import lax
from jax.experimental import pallas as pl
from jax.experimental.pallas import tpu as pltpu
from jax.experimental.pallas import tpu_sc as plsc

B = 64
N = 4096
E = 16384
IN_DIM = 12
HID = 64
P = B // 2
W2 = 2 * HID
BN2 = P * N

NC = 2
NS = 16
CH = 128
EPT = E // NS
NCH = EPT // CH
PPC = P // NC
RPT = N // NS


def _sc_agg_body(h_hbm, src_hbm, dst_hbm, zeros_hbm, out_hbm,
                 agg_sh, src_v, adj_v, dst_v, rows_v, zero_v, gsem, csem):
    cid = lax.axis_index("c")
    sid = lax.axis_index("s")
    pltpu.sync_copy(src_hbm.at[pl.ds(sid * NCH, NCH)], src_v)
    pltpu.sync_copy(dst_hbm.at[pl.ds(sid * NCH, NCH)], dst_v)
    pltpu.sync_copy(zeros_hbm, zero_v)

    my_rows = pl.ds(sid * RPT, RPT)

    def per_pair(g, carry):
        base = (cid * PPC + g) * N
        for r in range(NCH):
            for c in range(CH // 16):
                sl = pl.ds(c * 16, 16)
                adj_v[r, sl] = src_v[r, sl] + base
        @pl.when(g >= 1)
        def _():
            pltpu.make_async_copy(agg_sh.at[my_rows],
                                  out_hbm.at[pl.ds(base, RPT)], csem).wait()
        pltpu.sync_copy(zero_v, agg_sh.at[my_rows])
        plsc.subcore_barrier()
        gh = pltpu.async_copy(h_hbm.at[adj_v.at[0]], rows_v.at[0], gsem)
        for j in range(NCH):
            gh.wait()
            if j + 1 < NCH:
                gh = pltpu.async_copy(h_hbm.at[adj_v.at[j + 1]],
                                      rows_v.at[(j + 1) % 2], gsem)
            pltpu.sync_copy(rows_v.at[j % 2], agg_sh.at[dst_v.at[j]],
                            add=True)
        plsc.subcore_barrier()
        pltpu.async_copy(agg_sh.at[my_rows],
                         out_hbm.at[pl.ds(base + sid * RPT, RPT)], csem)
        return carry

    lax.fori_loop(0, PPC, per_pair, 0)
    pltpu.make_async_copy(agg_sh.at[my_rows],
                          out_hbm.at[pl.ds(0, RPT)], csem).wait()


@functools.cache
def _get_sc_agg():
    return pl.kernel(
        _sc_agg_body,
        out_type=jax.ShapeDtypeStruct((BN2, W2), jnp.float32),
        mesh=plsc.VectorSubcoreMesh(core_axis_name="c", subcore_axis_name="s"),
        scratch_types=[
            pltpu.VMEM_SHARED((N, W2), jnp.float32),
            pltpu.VMEM((NCH, CH), jnp.int32),
            pltpu.VMEM((NCH, CH), jnp.int32),
            pltpu.VMEM((NCH, CH), jnp.int32),
            pltpu.VMEM((2, CH, W2), jnp.float32),
            pltpu.VMEM((RPT, W2), jnp.float32),
            pltpu.SemaphoreType.DMA,
            pltpu.SemaphoreType.DMA,
        ],
    )


def _sc_deg_body(dst_hbm, ones_hbm, zeros_hbm, out_hbm,
                 deg_sh, dst_v, ones_v, zero_v):
    cid = lax.axis_index("c")
    sid = lax.axis_index("s")

    @pl.when(cid == 0)
    def _():
        pltpu.sync_copy(dst_hbm.at[pl.ds(sid * NCH, NCH)], dst_v)
        pltpu.sync_copy(ones_hbm, ones_v)
        pltpu.sync_copy(zeros_hbm, zero_v)
        pltpu.sync_copy(zero_v, deg_sh.at[pl.ds(sid * RPT, RPT)])
        plsc.subcore_barrier()
        for j in range(NCH):
            pltpu.sync_copy(ones_v, deg_sh.at[dst_v.at[j]], add=True)
        plsc.subcore_barrier()
        pltpu.sync_copy(deg_sh.at[pl.ds(sid * RPT, RPT)],
                        out_hbm.at[pl.ds(sid * RPT, RPT)])


@functools.cache
def _get_sc_deg():
    return pl.kernel(
        _sc_deg_body,
        out_type=jax.ShapeDtypeStruct((N, W2), jnp.float32),
        mesh=plsc.VectorSubcoreMesh(core_axis_name="c", subcore_axis_name="s"),
        scratch_types=[
            pltpu.VMEM_SHARED((N, W2), jnp.float32),
            pltpu.VMEM((NCH, CH), jnp.int32),
            pltpu.VMEM((CH, W2), jnp.float32),
            pltpu.VMEM((RPT, W2), jnp.float32),
        ],
    )


def _tc_in_kernel(x_ref, w_ref, b_ref, out_ref):
    h0 = lax.dot_general(x_ref[0, 0], w_ref[...], (((0,), (0,)), ((), ())),
                         preferred_element_type=jnp.float32)
    h1 = lax.dot_general(x_ref[0, 1], w_ref[...], (((0,), (0,)), ((), ())),
                         preferred_element_type=jnp.float32)
    out_ref[...] = jnp.concatenate([h0, h1], axis=1) + b_ref[...]


def _tc_input(xp, W_in, b2_row):
    return pl.pallas_call(
        _tc_in_kernel,
        grid=(P,),
        in_specs=[
            pl.BlockSpec((1, 2, IN_DIM, N), lambda i: (i, 0, 0, 0)),
            pl.BlockSpec((IN_DIM, HID), lambda i: (0, 0)),
            pl.BlockSpec((1, W2), lambda i: (0, 0)),
        ],
        out_specs=pl.BlockSpec((N, W2), lambda i: (i, 0)),
        out_shape=jax.ShapeDtypeStruct((BN2, W2), jnp.float32),
    )(xp, W_in, b2_row)


_BLK = 4096


def _half_ln(hn):
    m = jnp.mean(hn, axis=-1, keepdims=True)
    v = jnp.mean((hn - m) ** 2, axis=-1, keepdims=True)
    return (hn - m) / jnp.sqrt(v + 1e-5)


def _dense_core(h_ref, agg_ref, deg_ref, wr_ref, wn_ref, bc_ref,
                g_ref, be_ref):
    h = h_ref[...]
    a = agg_ref[...] / jnp.maximum(deg_ref[...], 1.0)
    hn = (lax.dot_general(h, wr_ref[...], (((1,), (0,)), ((), ())),
                          preferred_element_type=jnp.float32)
          + lax.dot_general(a, wn_ref[...], (((1,), (0,)), ((), ())),
                            preferred_element_type=jnp.float32)
          + bc_ref[...])
    ln = jnp.concatenate([_half_ln(hn[:, :HID]), _half_ln(hn[:, HID:])],
                         axis=1) * g_ref[...] + be_ref[...]
    return h + jnp.maximum(ln, 0.0)


def _tc_layer_kernel(h_ref, agg_ref, deg_ref, wr_ref, wn_ref, bc_ref,
                     g_ref, be_ref, out_ref):
    out_ref[...] = _dense_core(h_ref, agg_ref, deg_ref, wr_ref, wn_ref,
                               bc_ref, g_ref, be_ref)


def _tc_layer_head_kernel(h_ref, agg_ref, deg_ref, wr_ref, wn_ref, bc_ref,
                          g_ref, be_ref, wh_ref, bh_ref, out_ref):
    hf = _dense_core(h_ref, agg_ref, deg_ref, wr_ref, wn_ref,
                     bc_ref, g_ref, be_ref)
    out_ref[...] = (lax.dot_general(hf, wh_ref[...], (((1,), (0,)), ((), ())),
                                    preferred_element_type=jnp.float32)
                    + bh_ref[...])


def _dense_specs():
    return [
        pl.BlockSpec((_BLK, W2), lambda i: (i, 0)),
        pl.BlockSpec((_BLK, W2), lambda i: (i, 0)),
        pl.BlockSpec((N, 1), lambda i: (0, 0)),
        pl.BlockSpec((W2, W2), lambda i: (0, 0)),
        pl.BlockSpec((W2, W2), lambda i: (0, 0)),
        pl.BlockSpec((1, W2), lambda i: (0, 0)),
        pl.BlockSpec((1, W2), lambda i: (0, 0)),
        pl.BlockSpec((1, W2), lambda i: (0, 0)),
    ]


def _tc_layer(h, agg, deg, Wr2, Wn2, bc2, g2r, be2r):
    return pl.pallas_call(
        _tc_layer_kernel,
        grid=(BN2 // _BLK,),
        in_specs=_dense_specs(),
        out_specs=pl.BlockSpec((_BLK, W2), lambda i: (i, 0)),
        out_shape=jax.ShapeDtypeStruct((BN2, W2), jnp.float32),
    )(h, agg, deg, Wr2, Wn2, bc2, g2r, be2r)


def _tc_layer_head(h, agg, deg, Wr2, Wn2, bc2, g2r, be2r, wh2, bh):
    return pl.pallas_call(
        _tc_layer_head_kernel,
        grid=(BN2 // _BLK,),
        in_specs=_dense_specs() + [
            pl.BlockSpec((W2, 2), lambda i: (0, 0)),
            pl.BlockSpec((1, 2), lambda i: (0, 0)),
        ],
        out_specs=pl.BlockSpec((_BLK, 2), lambda i: (i, 0)),
        out_shape=jax.ShapeDtypeStruct((BN2, 2), jnp.float32),
    )(h, agg, deg, Wr2, Wn2, bc2, g2r, be2r, wh2, bh)


def kernel(x, edge_index, W_in, b_in, W_head, b_head,
           Wr0, Wn0, bc0, g0, be0,
           Wr1, Wn1, bc1, g1, be1,
           Wr2, Wn2, bc2, g2, be2):
    xp = x.reshape(P, 2, IN_DIM, N)
    src2d = edge_index[0].reshape(E // CH, CH)
    dst2d = edge_index[1].reshape(E // CH, CH)
    zeros_w2 = jnp.zeros((RPT, W2), jnp.float32)
    ones_w2 = jnp.ones((CH, W2), jnp.float32)
    eye2 = jnp.eye(2, dtype=jnp.float32)

    def pack_w(w):
        return jnp.kron(eye2, w)

    def pack_v(v):
        return jnp.tile(v.reshape(1, HID), (1, 2))

    h = _tc_input(xp, W_in, pack_v(b_in))
    deg = _get_sc_deg()(dst2d, ones_w2, zeros_w2)[:, :1]

    layers = [(Wr0, Wn0, bc0, g0, be0),
              (Wr1, Wn1, bc1, g1, be1),
              (Wr2, Wn2, bc2, g2, be2)]
    for i, (Wr, Wn, bc, g, be) in enumerate(layers):
        agg = _get_sc_agg()(h, src2d, dst2d, zeros_w2)
        args = (h, agg, deg, pack_w(Wr), pack_w(Wn), pack_v(bc),
                pack_v(g), pack_v(be))
        if i < 2:
            h = _tc_layer(*args)
        else:
            wh2 = jnp.concatenate(
                [jnp.concatenate([W_head, jnp.zeros((HID, 1), jnp.float32)], 1),
                 jnp.concatenate([jnp.zeros((HID, 1), jnp.float32), W_head], 1)],
                axis=0)
            logit2 = _tc_layer_head(*args, wh2,
                                    jnp.tile(b_head.reshape(1, 1), (1, 2)))
    return logit2.reshape(P, N, 2).transpose(0, 2, 1).reshape(B, 64, 64)


# speedup vs baseline: 1.2035x; 1.2035x over previous
import functools

import jax
import jax.numpy as jnp
from jax import lax
from jax.experimental import pallas as pl
from jax.experimental.pallas import tpu as pltpu
from jax.experimental.pallas import tpu_sc as plsc

B = 64
N = 4096
E = 16384
IN_DIM = 12
HID = 64
P = B // 2
W2 = 2 * HID
BN2 = P * N

NC = 2
NS = 16
CH = 128
EPT = E // NS
NCH = EPT // CH
PPC = P // NC
RPT = N // NS


def _sc_agg_body(h_hbm, src_hbm, dst_hbm, zeros_hbm, out_hbm,
                 agg_sh, src_v, adj_v, dst_v, rows_v, zero_v, gsem, csem):
    cid = lax.axis_index("c")
    sid = lax.axis_index("s")
    pltpu.sync_copy(src_hbm.at[pl.ds(sid * NCH, NCH)], src_v)
    pltpu.sync_copy(dst_hbm.at[pl.ds(sid * NCH, NCH)], dst_v)
    pltpu.sync_copy(zeros_hbm, zero_v)

    my_rows = pl.ds(sid * RPT, RPT)

    def per_pair(g, carry):
        base = (cid * PPC + g) * N
        for r in range(NCH):
            for c in range(CH // 16):
                sl = pl.ds(c * 16, 16)
                adj_v[r, sl] = src_v[r, sl] + base
        @pl.when(g >= 1)
        def _():
            pltpu.make_async_copy(agg_sh.at[my_rows],
                                  out_hbm.at[pl.ds(base, RPT)], csem).wait()
        pltpu.sync_copy(zero_v, agg_sh.at[my_rows])
        plsc.subcore_barrier()
        gh = pltpu.async_copy(h_hbm.at[adj_v.at[0]], rows_v.at[0], gsem)
        for j in range(NCH):
            gh.wait()
            if j + 1 < NCH:
                gh = pltpu.async_copy(h_hbm.at[adj_v.at[j + 1]],
                                      rows_v.at[(j + 1) % 2], gsem)
            pltpu.sync_copy(rows_v.at[j % 2], agg_sh.at[dst_v.at[j]],
                            add=True)
        plsc.subcore_barrier()
        pltpu.async_copy(agg_sh.at[my_rows],
                         out_hbm.at[pl.ds(base + sid * RPT, RPT)], csem)
        return carry

    lax.fori_loop(0, PPC, per_pair, 0)
    pltpu.make_async_copy(agg_sh.at[my_rows],
                          out_hbm.at[pl.ds(0, RPT)], csem).wait()


@functools.cache
def _get_sc_agg():
    return pl.kernel(
        _sc_agg_body,
        out_type=jax.ShapeDtypeStruct((BN2, W2), jnp.float32),
        mesh=plsc.VectorSubcoreMesh(core_axis_name="c", subcore_axis_name="s"),
        scratch_types=[
            pltpu.VMEM_SHARED((N, W2), jnp.float32),
            pltpu.VMEM((NCH, CH), jnp.int32),
            pltpu.VMEM((NCH, CH), jnp.int32),
            pltpu.VMEM((NCH, CH), jnp.int32),
            pltpu.VMEM((2, CH, W2), jnp.float32),
            pltpu.VMEM((RPT, W2), jnp.float32),
            pltpu.SemaphoreType.DMA,
            pltpu.SemaphoreType.DMA,
        ],
    )


def _sc_deg_body(dst_hbm, ones_hbm, zeros_hbm, out_hbm,
                 deg_sh, dst_v, ones_v, zero_v):
    cid = lax.axis_index("c")
    sid = lax.axis_index("s")

    @pl.when(cid == 0)
    def _():
        pltpu.sync_copy(dst_hbm.at[pl.ds(sid * NCH, NCH)], dst_v)
        pltpu.sync_copy(ones_hbm, ones_v)
        pltpu.sync_copy(zeros_hbm, zero_v)
        pltpu.sync_copy(zero_v, deg_sh.at[pl.ds(sid * RPT, RPT)])
        plsc.subcore_barrier()
        for j in range(NCH):
            pltpu.sync_copy(ones_v, deg_sh.at[dst_v.at[j]], add=True)
        plsc.subcore_barrier()
        pltpu.sync_copy(deg_sh.at[pl.ds(sid * RPT, RPT)],
                        out_hbm.at[pl.ds(sid * RPT, RPT)])


@functools.cache
def _get_sc_deg():
    return pl.kernel(
        _sc_deg_body,
        out_type=jax.ShapeDtypeStruct((N, W2), jnp.float32),
        mesh=plsc.VectorSubcoreMesh(core_axis_name="c", subcore_axis_name="s"),
        scratch_types=[
            pltpu.VMEM_SHARED((N, W2), jnp.float32),
            pltpu.VMEM((NCH, CH), jnp.int32),
            pltpu.VMEM((CH, W2), jnp.float32),
            pltpu.VMEM((RPT, W2), jnp.float32),
        ],
    )


def _tc_in_kernel(x_ref, w_ref, b_ref, out_ref):
    h0 = lax.dot_general(x_ref[0, 0], w_ref[...], (((0,), (0,)), ((), ())),
                         preferred_element_type=jnp.float32)
    h1 = lax.dot_general(x_ref[0, 1], w_ref[...], (((0,), (0,)), ((), ())),
                         preferred_element_type=jnp.float32)
    out_ref[...] = jnp.concatenate([h0, h1], axis=1) + b_ref[...]


def _tc_input(xp, W_in, b2_row):
    return pl.pallas_call(
        _tc_in_kernel,
        grid=(P,),
        in_specs=[
            pl.BlockSpec((1, 2, IN_DIM, N), lambda i: (i, 0, 0, 0)),
            pl.BlockSpec((IN_DIM, HID), lambda i: (0, 0)),
            pl.BlockSpec((1, W2), lambda i: (0, 0)),
        ],
        out_specs=pl.BlockSpec((N, W2), lambda i: (i, 0)),
        out_shape=jax.ShapeDtypeStruct((BN2, W2), jnp.float32),
    )(xp, W_in, b2_row)


_BLK = 4096


def _dense_core(h_ref, agg_ref, deg_ref, wr_ref, wn_ref, bc_ref,
                g_ref, be_ref, mh_ref):
    h = h_ref[...]
    a = agg_ref[...] / jnp.maximum(deg_ref[...], 1.0)
    hn = (lax.dot_general(h, wr_ref[...], (((1,), (0,)), ((), ())),
                          preferred_element_type=jnp.float32)
          + lax.dot_general(a, wn_ref[...], (((1,), (0,)), ((), ())),
                            preferred_element_type=jnp.float32)
          + bc_ref[...])
    m = lax.dot_general(hn, mh_ref[...], (((1,), (0,)), ((), ())),
                        preferred_element_type=jnp.float32)
    d = hn - m
    v = lax.dot_general(d * d, mh_ref[...], (((1,), (0,)), ((), ())),
                        preferred_element_type=jnp.float32)
    ln = d / jnp.sqrt(v + 1e-5) * g_ref[...] + be_ref[...]
    return h + jnp.maximum(ln, 0.0)


def _tc_layer_kernel(h_ref, agg_ref, deg_ref, wr_ref, wn_ref, bc_ref,
                     g_ref, be_ref, mh_ref, out_ref):
    out_ref[...] = _dense_core(h_ref, agg_ref, deg_ref, wr_ref, wn_ref,
                               bc_ref, g_ref, be_ref, mh_ref)


def _tc_layer_head_kernel(h_ref, agg_ref, deg_ref, wr_ref, wn_ref, bc_ref,
                          g_ref, be_ref, mh_ref, wh_ref, bh_ref, out_ref):
    hf = _dense_core(h_ref, agg_ref, deg_ref, wr_ref, wn_ref,
                     bc_ref, g_ref, be_ref, mh_ref)
    out_ref[...] = (lax.dot_general(hf, wh_ref[...], (((1,), (0,)), ((), ())),
                                    preferred_element_type=jnp.float32)
                    + bh_ref[...])


def _dense_specs():
    return [
        pl.BlockSpec((_BLK, W2), lambda i: (i, 0)),
        pl.BlockSpec((_BLK, W2), lambda i: (i, 0)),
        pl.BlockSpec((N, 1), lambda i: (0, 0)),
        pl.BlockSpec((W2, W2), lambda i: (0, 0)),
        pl.BlockSpec((W2, W2), lambda i: (0, 0)),
        pl.BlockSpec((1, W2), lambda i: (0, 0)),
        pl.BlockSpec((1, W2), lambda i: (0, 0)),
        pl.BlockSpec((1, W2), lambda i: (0, 0)),
        pl.BlockSpec((W2, W2), lambda i: (0, 0)),
    ]


def _tc_layer(h, agg, deg, Wr2, Wn2, bc2, g2r, be2r, mh):
    return pl.pallas_call(
        _tc_layer_kernel,
        grid=(BN2 // _BLK,),
        in_specs=_dense_specs(),
        out_specs=pl.BlockSpec((_BLK, W2), lambda i: (i, 0)),
        out_shape=jax.ShapeDtypeStruct((BN2, W2), jnp.float32),
    )(h, agg, deg, Wr2, Wn2, bc2, g2r, be2r, mh)


def _tc_layer_head(h, agg, deg, Wr2, Wn2, bc2, g2r, be2r, mh, wh2, bh):
    return pl.pallas_call(
        _tc_layer_head_kernel,
        grid=(BN2 // _BLK,),
        in_specs=_dense_specs() + [
            pl.BlockSpec((W2, 2), lambda i: (0, 0)),
            pl.BlockSpec((1, 2), lambda i: (0, 0)),
        ],
        out_specs=pl.BlockSpec((_BLK, 2), lambda i: (i, 0)),
        out_shape=jax.ShapeDtypeStruct((BN2, 2), jnp.float32),
    )(h, agg, deg, Wr2, Wn2, bc2, g2r, be2r, mh, wh2, bh)


def kernel(x, edge_index, W_in, b_in, W_head, b_head,
           Wr0, Wn0, bc0, g0, be0,
           Wr1, Wn1, bc1, g1, be1,
           Wr2, Wn2, bc2, g2, be2):
    xp = x.reshape(P, 2, IN_DIM, N)
    src2d = edge_index[0].reshape(E // CH, CH)
    dst2d = edge_index[1].reshape(E // CH, CH)
    zeros_w2 = jnp.zeros((RPT, W2), jnp.float32)
    ones_w2 = jnp.ones((CH, W2), jnp.float32)
    eye2 = jnp.eye(2, dtype=jnp.float32)
    mh = jnp.kron(eye2, jnp.full((HID, HID), 1.0 / HID, jnp.float32))

    def pack_w(w):
        return jnp.kron(eye2, w)

    def pack_v(v):
        return jnp.tile(v.reshape(1, HID), (1, 2))

    h = _tc_input(xp, W_in, pack_v(b_in))
    deg = _get_sc_deg()(dst2d, ones_w2, zeros_w2)[:, :1]

    layers = [(Wr0, Wn0, bc0, g0, be0),
              (Wr1, Wn1, bc1, g1, be1),
              (Wr2, Wn2, bc2, g2, be2)]
    for i, (Wr, Wn, bc, g, be) in enumerate(layers):
        agg = _get_sc_agg()(h, src2d, dst2d, zeros_w2)
        args = (h, agg, deg, pack_w(Wr), pack_w(Wn), pack_v(bc),
                pack_v(g), pack_v(be), mh)
        if i < 2:
            h = _tc_layer(*args)
        else:
            wh2 = jnp.concatenate(
                [jnp.concatenate([W_head, jnp.zeros((HID, 1), jnp.float32)], 1),
                 jnp.concatenate([jnp.zeros((HID, 1), jnp.float32), W_head], 1)],
                axis=0)
            logit2 = _tc_layer_head(*args, wh2,
                                    jnp.tile(b_head.reshape(1, 1), (1, 2)))
    return logit2.reshape(P, N, 2).transpose(0, 2, 1).reshape(B, 64, 64)


# speedup vs baseline: 1.2937x; 1.0750x over previous
import functools

import jax
import jax.numpy as jnp
from jax import lax
from jax.experimental import pallas as pl
from jax.experimental.pallas import tpu as pltpu
from jax.experimental.pallas import tpu_sc as plsc

B = 64
N = 4096
E = 16384
IN_DIM = 12
HID = 64
P = B // 2
W2 = 2 * HID
BN2 = P * N

NC = 2
NS = 16
CH = 128
EPT = E // NS
NCH = EPT // CH
RPT = N // NS

GRP = 4
PG = P // GRP
ROWS_G = PG * N
PPC = PG // NC


def _sc_agg_body(h_hbm, src_hbm, dst_hbm, zeros_hbm, out_hbm,
                 agg_sh, src_v, adj_v, dst_v, rows_v, zero_v, gsem, csem):
    cid = lax.axis_index("c")
    sid = lax.axis_index("s")
    pltpu.sync_copy(src_hbm.at[pl.ds(sid * NCH, NCH)], src_v)
    pltpu.sync_copy(dst_hbm.at[pl.ds(sid * NCH, NCH)], dst_v)
    pltpu.sync_copy(zeros_hbm, zero_v)

    my_rows = pl.ds(sid * RPT, RPT)

    def per_pair(g, carry):
        base = (cid * PPC + g) * N
        for r in range(NCH):
            for c in range(CH // 16):
                sl = pl.ds(c * 16, 16)
                adj_v[r, sl] = src_v[r, sl] + base
        @pl.when(g >= 1)
        def _():
            pltpu.make_async_copy(agg_sh.at[my_rows],
                                  out_hbm.at[pl.ds(base, RPT)], csem).wait()
        pltpu.sync_copy(zero_v, agg_sh.at[my_rows])
        plsc.subcore_barrier()
        gh = pltpu.async_copy(h_hbm.at[adj_v.at[0]], rows_v.at[0], gsem)
        for j in range(NCH):
            gh.wait()
            if j + 1 < NCH:
                gh = pltpu.async_copy(h_hbm.at[adj_v.at[j + 1]],
                                      rows_v.at[(j + 1) % 2], gsem)
            pltpu.sync_copy(rows_v.at[j % 2], agg_sh.at[dst_v.at[j]],
                            add=True)
        plsc.subcore_barrier()
        pltpu.async_copy(agg_sh.at[my_rows],
                         out_hbm.at[pl.ds(base + sid * RPT, RPT)], csem)
        return carry

    lax.fori_loop(0, PPC, per_pair, 0)
    pltpu.make_async_copy(agg_sh.at[my_rows],
                          out_hbm.at[pl.ds(0, RPT)], csem).wait()


@functools.cache
def _get_sc_agg():
    return pl.kernel(
        _sc_agg_body,
        out_type=jax.ShapeDtypeStruct((ROWS_G, W2), jnp.float32),
        mesh=plsc.VectorSubcoreMesh(core_axis_name="c", subcore_axis_name="s"),
        scratch_types=[
            pltpu.VMEM_SHARED((N, W2), jnp.float32),
            pltpu.VMEM((NCH, CH), jnp.int32),
            pltpu.VMEM((NCH, CH), jnp.int32),
            pltpu.VMEM((NCH, CH), jnp.int32),
            pltpu.VMEM((2, CH, W2), jnp.float32),
            pltpu.VMEM((RPT, W2), jnp.float32),
            pltpu.SemaphoreType.DMA,
            pltpu.SemaphoreType.DMA,
        ],
    )


def _sc_deg_body(dst_hbm, ones_hbm, zeros_hbm, out_hbm,
                 deg_sh, dst_v, ones_v, zero_v):
    cid = lax.axis_index("c")
    sid = lax.axis_index("s")

    @pl.when(cid == 0)
    def _():
        pltpu.sync_copy(dst_hbm.at[pl.ds(sid * NCH, NCH)], dst_v)
        pltpu.sync_copy(ones_hbm, ones_v)
        pltpu.sync_copy(zeros_hbm, zero_v)
        pltpu.sync_copy(zero_v, deg_sh.at[pl.ds(sid * RPT, RPT)])
        plsc.subcore_barrier()
        for j in range(NCH):
            pltpu.sync_copy(ones_v, deg_sh.at[dst_v.at[j]], add=True)
        plsc.subcore_barrier()
        pltpu.sync_copy(deg_sh.at[pl.ds(sid * RPT, RPT)],
                        out_hbm.at[pl.ds(sid * RPT, RPT)])


@functools.cache
def _get_sc_deg():
    return pl.kernel(
        _sc_deg_body,
        out_type=jax.ShapeDtypeStruct((N, W2), jnp.float32),
        mesh=plsc.VectorSubcoreMesh(core_axis_name="c", subcore_axis_name="s"),
        scratch_types=[
            pltpu.VMEM_SHARED((N, W2), jnp.float32),
            pltpu.VMEM((NCH, CH), jnp.int32),
            pltpu.VMEM((CH, W2), jnp.float32),
            pltpu.VMEM((RPT, W2), jnp.float32),
        ],
    )


def _tc_in_kernel(x_ref, w_ref, b_ref, out_ref):
    h0 = lax.dot_general(x_ref[0, 0], w_ref[...], (((0,), (0,)), ((), ())),
                         preferred_element_type=jnp.float32)
    h1 = lax.dot_general(x_ref[0, 1], w_ref[...], (((0,), (0,)), ((), ())),
                         preferred_element_type=jnp.float32)
    out_ref[...] = jnp.concatenate([h0, h1], axis=1) + b_ref[...]


def _tc_input(xp, W_in, b2_row, k):
    return pl.pallas_call(
        _tc_in_kernel,
        grid=(PG,),
        in_specs=[
            pl.BlockSpec((1, 2, IN_DIM, N), lambda i: (k * PG + i, 0, 0, 0)),
            pl.BlockSpec((IN_DIM, HID), lambda i: (0, 0)),
            pl.BlockSpec((1, W2), lambda i: (0, 0)),
        ],
        out_specs=pl.BlockSpec((N, W2), lambda i: (i, 0)),
        out_shape=jax.ShapeDtypeStruct((ROWS_G, W2), jnp.float32),
    )(xp, W_in, b2_row)


_BLK = 4096


def _dense_core(h_ref, agg_ref, deg_ref, wr_ref, wn_ref, bc_ref,
                g_ref, be_ref, mh_ref):
    h = h_ref[...]
    a = agg_ref[...] / jnp.maximum(deg_ref[...], 1.0)
    hn = (lax.dot_general(h, wr_ref[...], (((1,), (0,)), ((), ())),
                          preferred_element_type=jnp.float32)
          + lax.dot_general(a, wn_ref[...], (((1,), (0,)), ((), ())),
                            preferred_element_type=jnp.float32)
          + bc_ref[...])
    m = lax.dot_general(hn, mh_ref[...], (((1,), (0,)), ((), ())),
                        preferred_element_type=jnp.float32)
    d = hn - m
    v = lax.dot_general(d * d, mh_ref[...], (((1,), (0,)), ((), ())),
                        preferred_element_type=jnp.float32)
    ln = d / jnp.sqrt(v + 1e-5) * g_ref[...] + be_ref[...]
    return h + jnp.maximum(ln, 0.0)


def _tc_layer_kernel(h_ref, agg_ref, deg_ref, wr_ref, wn_ref, bc_ref,
                     g_ref, be_ref, mh_ref, out_ref):
    out_ref[...] = _dense_core(h_ref, agg_ref, deg_ref, wr_ref, wn_ref,
                               bc_ref, g_ref, be_ref, mh_ref)


def _tc_layer_head_kernel(h_ref, agg_ref, deg_ref, wr_ref, wn_ref, bc_ref,
                          g_ref, be_ref, mh_ref, wh_ref, bh_ref, out_ref):
    hf = _dense_core(h_ref, agg_ref, deg_ref, wr_ref, wn_ref,
                     bc_ref, g_ref, be_ref, mh_ref)
    out_ref[...] = (lax.dot_general(hf, wh_ref[...], (((1,), (0,)), ((), ())),
                                    preferred_element_type=jnp.float32)
                    + bh_ref[...])


def _dense_specs():
    return [
        pl.BlockSpec((_BLK, W2), lambda i: (i, 0)),
        pl.BlockSpec((_BLK, W2), lambda i: (i, 0)),
        pl.BlockSpec((N, 1), lambda i: (0, 0)),
        pl.BlockSpec((W2, W2), lambda i: (0, 0)),
        pl.BlockSpec((W2, W2), lambda i: (0, 0)),
        pl.BlockSpec((1, W2), lambda i: (0, 0)),
        pl.BlockSpec((1, W2), lambda i: (0, 0)),
        pl.BlockSpec((1, W2), lambda i: (0, 0)),
        pl.BlockSpec((W2, W2), lambda i: (0, 0)),
    ]


def _tc_layer(h, agg, deg, Wr2, Wn2, bc2, g2r, be2r, mh):
    return pl.pallas_call(
        _tc_layer_kernel,
        grid=(ROWS_G // _BLK,),
        in_specs=_dense_specs(),
        out_specs=pl.BlockSpec((_BLK, W2), lambda i: (i, 0)),
        out_shape=jax.ShapeDtypeStruct((ROWS_G, W2), jnp.float32),
    )(h, agg, deg, Wr2, Wn2, bc2, g2r, be2r, mh)


def _tc_layer_head(h, agg, deg, Wr2, Wn2, bc2, g2r, be2r, mh, wh2, bh):
    return pl.pallas_call(
        _tc_layer_head_kernel,
        grid=(ROWS_G // _BLK,),
        in_specs=_dense_specs() + [
            pl.BlockSpec((W2, 2), lambda i: (0, 0)),
            pl.BlockSpec((1, 2), lambda i: (0, 0)),
        ],
        out_specs=pl.BlockSpec((_BLK, 2), lambda i: (i, 0)),
        out_shape=jax.ShapeDtypeStruct((ROWS_G, 2), jnp.float32),
    )(h, agg, deg, Wr2, Wn2, bc2, g2r, be2r, mh, wh2, bh)


def kernel(x, edge_index, W_in, b_in, W_head, b_head,
           Wr0, Wn0, bc0, g0, be0,
           Wr1, Wn1, bc1, g1, be1,
           Wr2, Wn2, bc2, g2, be2):
    xp = x.reshape(P, 2, IN_DIM, N)
    src2d = edge_index[0].reshape(E // CH, CH)
    dst2d = edge_index[1].reshape(E // CH, CH)
    zeros_w2 = jnp.zeros((RPT, W2), jnp.float32)
    ones_w2 = jnp.ones((CH, W2), jnp.float32)
    eye2 = jnp.eye(2, dtype=jnp.float32)
    mh = jnp.kron(eye2, jnp.full((HID, HID), 1.0 / HID, jnp.float32))

    def pack_w(w):
        return jnp.kron(eye2, w)

    def pack_v(v):
        return jnp.tile(v.reshape(1, HID), (1, 2))

    hs = [_tc_input(xp, W_in, pack_v(b_in), k) for k in range(GRP)]
    deg = _get_sc_deg()(dst2d, ones_w2, zeros_w2)[:, :1]
    wh2 = jnp.concatenate(
        [jnp.concatenate([W_head, jnp.zeros((HID, 1), jnp.float32)], 1),
         jnp.concatenate([jnp.zeros((HID, 1), jnp.float32), W_head], 1)],
        axis=0)
    bh2 = jnp.tile(b_head.reshape(1, 1), (1, 2))

    layers = [(Wr0, Wn0, bc0, g0, be0),
              (Wr1, Wn1, bc1, g1, be1),
              (Wr2, Wn2, bc2, g2, be2)]
    for i, (Wr, Wn, bc, g, be) in enumerate(layers):
        aggs = [_get_sc_agg()(hk, src2d, dst2d, zeros_w2) for hk in hs]
        packed = (pack_w(Wr), pack_w(Wn), pack_v(bc), pack_v(g), pack_v(be))
        if i < 2:
            hs = [_tc_layer(hs[k], aggs[k], deg, *packed, mh)
                  for k in range(GRP)]
        else:
            logit2 = jnp.concatenate(
                [_tc_layer_head(hs[k], aggs[k], deg, *packed, mh, wh2, bh2)
                 for k in range(GRP)], axis=0)
    return logit2.reshape(P, N, 2).transpose(0, 2, 1).reshape(B, 64, 64)


# speedup vs baseline: 1.5673x; 1.2114x over previous
import functools

import jax
import jax.numpy as jnp
from jax import lax
from jax.experimental import pallas as pl
from jax.experimental.pallas import tpu as pltpu
from jax.experimental.pallas import tpu_sc as plsc

B = 64
N = 4096
E = 16384
IN_DIM = 12
HID = 64
P = B // 2
W2 = 2 * HID
BN2 = P * N

NC = 2
NS = 16
CH = 128
EPT = E // NS
NCH = EPT // CH
RPT = N // NS
ZR = 64

GRP = 4
PG = P // GRP
ROWS_G = PG * N
PPC = PG // NC


def _sc_agg_body(h_hbm, src_hbm, dst_hbm, zeros_hbm, out_hbm,
                 agg_sh, src_v, adj_v, dst_v, rows_v, zero_v, gsem, ssem,
                 csem):
    cid = lax.axis_index("c")
    sid = lax.axis_index("s")
    pltpu.sync_copy(src_hbm.at[pl.ds(sid * NCH, NCH)], src_v)
    pltpu.sync_copy(dst_hbm.at[pl.ds(sid * NCH, NCH)], dst_v)
    pltpu.sync_copy(zeros_hbm, zero_v)

    my_rows = pl.ds(sid * RPT, RPT)

    def per_pair(g, carry):
        base = (cid * PPC + g) * N
        for r in range(NCH):
            for c in range(CH // 16):
                sl = pl.ds(c * 16, 16)
                adj_v[r, sl] = src_v[r, sl] + base
        @pl.when(g >= 1)
        def _():
            pltpu.make_async_copy(agg_sh.at[my_rows],
                                  out_hbm.at[pl.ds(base, RPT)], csem).wait()
        for z in range(RPT // ZR):
            pltpu.sync_copy(zero_v,
                            agg_sh.at[pl.ds(sid * RPT + z * ZR, ZR)])
        plsc.subcore_barrier()
        pltpu.async_copy(h_hbm.at[adj_v.at[0]], rows_v.at[0], gsem)
        for j in range(NCH):
            b = j % 4
            if j + 1 < NCH:
                if j + 1 >= 4:
                    pltpu.make_async_copy(rows_v.at[(j + 1) % 4],
                                          agg_sh.at[dst_v.at[j]], ssem).wait()
                pltpu.async_copy(h_hbm.at[adj_v.at[j + 1]],
                                 rows_v.at[(j + 1) % 4], gsem)
            pltpu.make_async_copy(h_hbm.at[adj_v.at[j]], rows_v.at[b],
                                  gsem).wait()
            pltpu.async_copy(rows_v.at[b], agg_sh.at[dst_v.at[j]], ssem,
                             add=True)
        for _ in range(4):
            pltpu.make_async_copy(rows_v.at[0], agg_sh.at[dst_v.at[0]],
                                  ssem).wait()
        plsc.subcore_barrier()
        pltpu.async_copy(agg_sh.at[my_rows],
                         out_hbm.at[pl.ds(base + sid * RPT, RPT)], csem)
        return carry

    lax.fori_loop(0, PPC, per_pair, 0)
    pltpu.make_async_copy(agg_sh.at[my_rows],
                          out_hbm.at[pl.ds(0, RPT)], csem).wait()


@functools.cache
def _get_sc_agg():
    return pl.kernel(
        _sc_agg_body,
        out_type=jax.ShapeDtypeStruct((ROWS_G, W2), jnp.float32),
        mesh=plsc.VectorSubcoreMesh(core_axis_name="c", subcore_axis_name="s"),
        scratch_types=[
            pltpu.VMEM_SHARED((N, W2), jnp.float32),
            pltpu.VMEM((NCH, CH), jnp.int32),
            pltpu.VMEM((NCH, CH), jnp.int32),
            pltpu.VMEM((NCH, CH), jnp.int32),
            pltpu.VMEM((4, CH, W2), jnp.float32),
            pltpu.VMEM((ZR, W2), jnp.float32),
            pltpu.SemaphoreType.DMA,
            pltpu.SemaphoreType.DMA,
            pltpu.SemaphoreType.DMA,
        ],
    )


def _sc_deg_body(dst_hbm, ones_hbm, zeros_hbm, out_hbm,
                 deg_sh, dst_v, ones_v, zero_v):
    cid = lax.axis_index("c")
    sid = lax.axis_index("s")

    @pl.when(cid == 0)
    def _():
        pltpu.sync_copy(dst_hbm.at[pl.ds(sid * NCH, NCH)], dst_v)
        pltpu.sync_copy(ones_hbm, ones_v)
        pltpu.sync_copy(zeros_hbm, zero_v)
        for z in range(RPT // ZR):
            pltpu.sync_copy(zero_v,
                            deg_sh.at[pl.ds(sid * RPT + z * ZR, ZR)])
        plsc.subcore_barrier()
        for j in range(NCH):
            pltpu.sync_copy(ones_v, deg_sh.at[dst_v.at[j]], add=True)
        plsc.subcore_barrier()
        pltpu.sync_copy(deg_sh.at[pl.ds(sid * RPT, RPT)],
                        out_hbm.at[pl.ds(sid * RPT, RPT)])


@functools.cache
def _get_sc_deg():
    return pl.kernel(
        _sc_deg_body,
        out_type=jax.ShapeDtypeStruct((N, W2), jnp.float32),
        mesh=plsc.VectorSubcoreMesh(core_axis_name="c", subcore_axis_name="s"),
        scratch_types=[
            pltpu.VMEM_SHARED((N, W2), jnp.float32),
            pltpu.VMEM((NCH, CH), jnp.int32),
            pltpu.VMEM((CH, W2), jnp.float32),
            pltpu.VMEM((ZR, W2), jnp.float32),
        ],
    )


def _tc_in_kernel(x_ref, w_ref, b_ref, out_ref):
    h0 = lax.dot_general(x_ref[0, 0], w_ref[...], (((0,), (0,)), ((), ())),
                         preferred_element_type=jnp.float32)
    h1 = lax.dot_general(x_ref[0, 1], w_ref[...], (((0,), (0,)), ((), ())),
                         preferred_element_type=jnp.float32)
    out_ref[...] = jnp.concatenate([h0, h1], axis=1) + b_ref[...]


def _tc_input(xp, W_in, b2_row, k):
    return pl.pallas_call(
        _tc_in_kernel,
        grid=(PG,),
        in_specs=[
            pl.BlockSpec((1, 2, IN_DIM, N), lambda i: (k * PG + i, 0, 0, 0)),
            pl.BlockSpec((IN_DIM, HID), lambda i: (0, 0)),
            pl.BlockSpec((1, W2), lambda i: (0, 0)),
        ],
        out_specs=pl.BlockSpec((N, W2), lambda i: (i, 0)),
        out_shape=jax.ShapeDtypeStruct((ROWS_G, W2), jnp.float32),
    )(xp, W_in, b2_row)


_BLK = 4096


def _dense_core(h_ref, agg_ref, deg_ref, wr_ref, wn_ref, bc_ref,
                g_ref, be_ref, mh_ref):
    h = h_ref[...]
    a = agg_ref[...] / jnp.maximum(deg_ref[...], 1.0)
    hn = (lax.dot_general(h, wr_ref[...], (((1,), (0,)), ((), ())),
                          preferred_element_type=jnp.float32)
          + lax.dot_general(a, wn_ref[...], (((1,), (0,)), ((), ())),
                            preferred_element_type=jnp.float32)
          + bc_ref[...])
    m = lax.dot_general(hn, mh_ref[...], (((1,), (0,)), ((), ())),
                        preferred_element_type=jnp.float32)
    d = hn - m
    v = lax.dot_general(d * d, mh_ref[...], (((1,), (0,)), ((), ())),
                        preferred_element_type=jnp.float32)
    ln = d / jnp.sqrt(v + 1e-5) * g_ref[...] + be_ref[...]
    return h + jnp.maximum(ln, 0.0)


def _tc_layer_kernel(h_ref, agg_ref, deg_ref, wr_ref, wn_ref, bc_ref,
                     g_ref, be_ref, mh_ref, out_ref):
    out_ref[...] = _dense_core(h_ref, agg_ref, deg_ref, wr_ref, wn_ref,
                               bc_ref, g_ref, be_ref, mh_ref)


def _tc_layer_head_kernel(h_ref, agg_ref, deg_ref, wr_ref, wn_ref, bc_ref,
                          g_ref, be_ref, mh_ref, wh_ref, bh_ref, out_ref):
    hf = _dense_core(h_ref, agg_ref, deg_ref, wr_ref, wn_ref,
                     bc_ref, g_ref, be_ref, mh_ref)
    out_ref[...] = (lax.dot_general(hf, wh_ref[...], (((1,), (0,)), ((), ())),
                                    preferred_element_type=jnp.float32)
                    + bh_ref[...])


def _dense_specs():
    return [
        pl.BlockSpec((_BLK, W2), lambda i: (i, 0)),
        pl.BlockSpec((_BLK, W2), lambda i: (i, 0)),
        pl.BlockSpec((N, 1), lambda i: (0, 0)),
        pl.BlockSpec((W2, W2), lambda i: (0, 0)),
        pl.BlockSpec((W2, W2), lambda i: (0, 0)),
        pl.BlockSpec((1, W2), lambda i: (0, 0)),
        pl.BlockSpec((1, W2), lambda i: (0, 0)),
        pl.BlockSpec((1, W2), lambda i: (0, 0)),
        pl.BlockSpec((W2, W2), lambda i: (0, 0)),
    ]


def _tc_layer(h, agg, deg, Wr2, Wn2, bc2, g2r, be2r, mh):
    return pl.pallas_call(
        _tc_layer_kernel,
        grid=(ROWS_G // _BLK,),
        in_specs=_dense_specs(),
        out_specs=pl.BlockSpec((_BLK, W2), lambda i: (i, 0)),
        out_shape=jax.ShapeDtypeStruct((ROWS_G, W2), jnp.float32),
    )(h, agg, deg, Wr2, Wn2, bc2, g2r, be2r, mh)


def _tc_layer_head(h, agg, deg, Wr2, Wn2, bc2, g2r, be2r, mh, wh2, bh):
    return pl.pallas_call(
        _tc_layer_head_kernel,
        grid=(ROWS_G // _BLK,),
        in_specs=_dense_specs() + [
            pl.BlockSpec((W2, 2), lambda i: (0, 0)),
            pl.BlockSpec((1, 2), lambda i: (0, 0)),
        ],
        out_specs=pl.BlockSpec((_BLK, 2), lambda i: (i, 0)),
        out_shape=jax.ShapeDtypeStruct((ROWS_G, 2), jnp.float32),
    )(h, agg, deg, Wr2, Wn2, bc2, g2r, be2r, mh, wh2, bh)


def kernel(x, edge_index, W_in, b_in, W_head, b_head,
           Wr0, Wn0, bc0, g0, be0,
           Wr1, Wn1, bc1, g1, be1,
           Wr2, Wn2, bc2, g2, be2):
    xp = x.reshape(P, 2, IN_DIM, N)
    src2d = edge_index[0].reshape(E // CH, CH)
    dst2d = edge_index[1].reshape(E // CH, CH)
    zeros_w2 = jnp.zeros((ZR, W2), jnp.float32)
    ones_w2 = jnp.ones((CH, W2), jnp.float32)
    eye2 = jnp.eye(2, dtype=jnp.float32)
    mh = jnp.kron(eye2, jnp.full((HID, HID), 1.0 / HID, jnp.float32))

    def pack_w(w):
        return jnp.kron(eye2, w)

    def pack_v(v):
        return jnp.tile(v.reshape(1, HID), (1, 2))

    hs = [_tc_input(xp, W_in, pack_v(b_in), k) for k in range(GRP)]
    deg = _get_sc_deg()(dst2d, ones_w2, zeros_w2)[:, :1]
    wh2 = jnp.concatenate(
        [jnp.concatenate([W_head, jnp.zeros((HID, 1), jnp.float32)], 1),
         jnp.concatenate([jnp.zeros((HID, 1), jnp.float32), W_head], 1)],
        axis=0)
    bh2 = jnp.tile(b_head.reshape(1, 1), (1, 2))

    layers = [(Wr0, Wn0, bc0, g0, be0),
              (Wr1, Wn1, bc1, g1, be1),
              (Wr2, Wn2, bc2, g2, be2)]
    for i, (Wr, Wn, bc, g, be) in enumerate(layers):
        aggs = [_get_sc_agg()(hk, src2d, dst2d, zeros_w2) for hk in hs]
        packed = (pack_w(Wr), pack_w(Wn), pack_v(bc), pack_v(g), pack_v(be))
        if i < 2:
            hs = [_tc_layer(hs[k], aggs[k], deg, *packed, mh)
                  for k in range(GRP)]
        else:
            logit2 = jnp.concatenate(
                [_tc_layer_head(hs[k], aggs[k], deg, *packed, mh, wh2, bh2)
                 for k in range(GRP)], axis=0)
    return logit2.reshape(P, N, 2).transpose(0, 2, 1).reshape(B, 64, 64)


# speedup vs baseline: 1.5904x; 1.0147x over previous
import functools

import jax
import jax.numpy as jnp
from jax import lax
from jax.experimental import pallas as pl
from jax.experimental.pallas import tpu as pltpu
from jax.experimental.pallas import tpu_sc as plsc

B = 64
N = 4096
E = 16384
IN_DIM = 12
HID = 64
P = B // 2
W2 = 2 * HID
BN2 = P * N

NC = 2
NS = 16
CH = 128
EPT = E // NS
NCH = EPT // CH
RPT = N // NS
ZR = 64

GRP = 2
PG = P // GRP
ROWS_G = PG * N
PPC = PG // NC


def _sc_agg_body(h_hbm, src_hbm, dst_hbm, zeros_hbm, out_hbm,
                 agg_sh, src_v, adj_v, dst_v, rows_v, zero_v, gsem, ssem,
                 csem):
    cid = lax.axis_index("c")
    sid = lax.axis_index("s")
    pltpu.sync_copy(src_hbm.at[pl.ds(sid * NCH, NCH)], src_v)
    pltpu.sync_copy(dst_hbm.at[pl.ds(sid * NCH, NCH)], dst_v)
    pltpu.sync_copy(zeros_hbm, zero_v)

    my_rows = pl.ds(sid * RPT, RPT)

    def per_pair(g, carry):
        base = (cid * PPC + g) * N
        for r in range(NCH):
            for c in range(CH // 16):
                sl = pl.ds(c * 16, 16)
                adj_v[r, sl] = src_v[r, sl] + base
        @pl.when(g >= 1)
        def _():
            pltpu.make_async_copy(agg_sh.at[my_rows],
                                  out_hbm.at[pl.ds(base, RPT)], csem).wait()
        for z in range(RPT // ZR):
            pltpu.sync_copy(zero_v,
                            agg_sh.at[pl.ds(sid * RPT + z * ZR, ZR)])
        plsc.subcore_barrier()
        pltpu.async_copy(h_hbm.at[adj_v.at[0]], rows_v.at[0], gsem)
        for j in range(NCH):
            b = j % 4
            if j + 1 < NCH:
                if j + 1 >= 4:
                    pltpu.make_async_copy(rows_v.at[(j + 1) % 4],
                                          agg_sh.at[dst_v.at[j]], ssem).wait()
                pltpu.async_copy(h_hbm.at[adj_v.at[j + 1]],
                                 rows_v.at[(j + 1) % 4], gsem)
            pltpu.make_async_copy(h_hbm.at[adj_v.at[j]], rows_v.at[b],
                                  gsem).wait()
            pltpu.async_copy(rows_v.at[b], agg_sh.at[dst_v.at[j]], ssem,
                             add=True)
        for _ in range(4):
            pltpu.make_async_copy(rows_v.at[0], agg_sh.at[dst_v.at[0]],
                                  ssem).wait()
        plsc.subcore_barrier()
        pltpu.async_copy(agg_sh.at[my_rows],
                         out_hbm.at[pl.ds(base + sid * RPT, RPT)], csem)
        return carry

    lax.fori_loop(0, PPC, per_pair, 0)
    pltpu.make_async_copy(agg_sh.at[my_rows],
                          out_hbm.at[pl.ds(0, RPT)], csem).wait()


@functools.cache
def _get_sc_agg():
    return pl.kernel(
        _sc_agg_body,
        out_type=jax.ShapeDtypeStruct((ROWS_G, W2), jnp.float32),
        mesh=plsc.VectorSubcoreMesh(core_axis_name="c", subcore_axis_name="s"),
        scratch_types=[
            pltpu.VMEM_SHARED((N, W2), jnp.float32),
            pltpu.VMEM((NCH, CH), jnp.int32),
            pltpu.VMEM((NCH, CH), jnp.int32),
            pltpu.VMEM((NCH, CH), jnp.int32),
            pltpu.VMEM((4, CH, W2), jnp.float32),
            pltpu.VMEM((ZR, W2), jnp.float32),
            pltpu.SemaphoreType.DMA,
            pltpu.SemaphoreType.DMA,
            pltpu.SemaphoreType.DMA,
        ],
    )


def _sc_deg_body(dst_hbm, ones_hbm, zeros_hbm, out_hbm,
                 deg_sh, dst_v, ones_v, zero_v):
    cid = lax.axis_index("c")
    sid = lax.axis_index("s")

    @pl.when(cid == 0)
    def _():
        pltpu.sync_copy(dst_hbm.at[pl.ds(sid * NCH, NCH)], dst_v)
        pltpu.sync_copy(ones_hbm, ones_v)
        pltpu.sync_copy(zeros_hbm, zero_v)
        for z in range(RPT // ZR):
            pltpu.sync_copy(zero_v,
                            deg_sh.at[pl.ds(sid * RPT + z * ZR, ZR)])
        plsc.subcore_barrier()
        for j in range(NCH):
            pltpu.sync_copy(ones_v, deg_sh.at[dst_v.at[j]], add=True)
        plsc.subcore_barrier()
        pltpu.sync_copy(deg_sh.at[pl.ds(sid * RPT, RPT)],
                        out_hbm.at[pl.ds(sid * RPT, RPT)])


@functools.cache
def _get_sc_deg():
    return pl.kernel(
        _sc_deg_body,
        out_type=jax.ShapeDtypeStruct((N, W2), jnp.float32),
        mesh=plsc.VectorSubcoreMesh(core_axis_name="c", subcore_axis_name="s"),
        scratch_types=[
            pltpu.VMEM_SHARED((N, W2), jnp.float32),
            pltpu.VMEM((NCH, CH), jnp.int32),
            pltpu.VMEM((CH, W2), jnp.float32),
            pltpu.VMEM((ZR, W2), jnp.float32),
        ],
    )


def _tc_in_kernel(x_ref, w_ref, b_ref, out_ref):
    h0 = lax.dot_general(x_ref[0, 0], w_ref[...], (((0,), (0,)), ((), ())),
                         preferred_element_type=jnp.float32)
    h1 = lax.dot_general(x_ref[0, 1], w_ref[...], (((0,), (0,)), ((), ())),
                         preferred_element_type=jnp.float32)
    out_ref[...] = jnp.concatenate([h0, h1], axis=1) + b_ref[...]


def _tc_input(xp, W_in, b2_row, k):
    return pl.pallas_call(
        _tc_in_kernel,
        grid=(PG,),
        in_specs=[
            pl.BlockSpec((1, 2, IN_DIM, N), lambda i: (k * PG + i, 0, 0, 0)),
            pl.BlockSpec((IN_DIM, HID), lambda i: (0, 0)),
            pl.BlockSpec((1, W2), lambda i: (0, 0)),
        ],
        out_specs=pl.BlockSpec((N, W2), lambda i: (i, 0)),
        out_shape=jax.ShapeDtypeStruct((ROWS_G, W2), jnp.float32),
    )(xp, W_in, b2_row)


_BLK = 4096


def _dense_core(h_ref, agg_ref, deg_ref, wr_ref, wn_ref, bc_ref,
                g_ref, be_ref, mh_ref):
    h = h_ref[...]
    a = agg_ref[...] / jnp.maximum(deg_ref[...], 1.0)
    hn = (lax.dot_general(h, wr_ref[...], (((1,), (0,)), ((), ())),
                          preferred_element_type=jnp.float32)
          + lax.dot_general(a, wn_ref[...], (((1,), (0,)), ((), ())),
                            preferred_element_type=jnp.float32)
          + bc_ref[...])
    m = lax.dot_general(hn, mh_ref[...], (((1,), (0,)), ((), ())),
                        preferred_element_type=jnp.float32)
    d = hn - m
    v = lax.dot_general(d * d, mh_ref[...], (((1,), (0,)), ((), ())),
                        preferred_element_type=jnp.float32)
    ln = d / jnp.sqrt(v + 1e-5) * g_ref[...] + be_ref[...]
    return h + jnp.maximum(ln, 0.0)


def _tc_layer_kernel(h_ref, agg_ref, deg_ref, wr_ref, wn_ref, bc_ref,
                     g_ref, be_ref, mh_ref, out_ref):
    out_ref[...] = _dense_core(h_ref, agg_ref, deg_ref, wr_ref, wn_ref,
                               bc_ref, g_ref, be_ref, mh_ref)


def _tc_layer_head_kernel(h_ref, agg_ref, deg_ref, wr_ref, wn_ref, bc_ref,
                          g_ref, be_ref, mh_ref, wh_ref, bh_ref, out_ref):
    hf = _dense_core(h_ref, agg_ref, deg_ref, wr_ref, wn_ref,
                     bc_ref, g_ref, be_ref, mh_ref)
    out_ref[...] = (lax.dot_general(hf, wh_ref[...], (((1,), (0,)), ((), ())),
                                    preferred_element_type=jnp.float32)
                    + bh_ref[...])


def _dense_specs():
    return [
        pl.BlockSpec((_BLK, W2), lambda i: (i, 0)),
        pl.BlockSpec((_BLK, W2), lambda i: (i, 0)),
        pl.BlockSpec((N, 1), lambda i: (0, 0)),
        pl.BlockSpec((W2, W2), lambda i: (0, 0)),
        pl.BlockSpec((W2, W2), lambda i: (0, 0)),
        pl.BlockSpec((1, W2), lambda i: (0, 0)),
        pl.BlockSpec((1, W2), lambda i: (0, 0)),
        pl.BlockSpec((1, W2), lambda i: (0, 0)),
        pl.BlockSpec((W2, W2), lambda i: (0, 0)),
    ]


def _tc_layer(h, agg, deg, Wr2, Wn2, bc2, g2r, be2r, mh):
    return pl.pallas_call(
        _tc_layer_kernel,
        grid=(ROWS_G // _BLK,),
        in_specs=_dense_specs(),
        out_specs=pl.BlockSpec((_BLK, W2), lambda i: (i, 0)),
        out_shape=jax.ShapeDtypeStruct((ROWS_G, W2), jnp.float32),
    )(h, agg, deg, Wr2, Wn2, bc2, g2r, be2r, mh)


def _tc_layer_head(h, agg, deg, Wr2, Wn2, bc2, g2r, be2r, mh, wh2, bh):
    return pl.pallas_call(
        _tc_layer_head_kernel,
        grid=(ROWS_G // _BLK,),
        in_specs=_dense_specs() + [
            pl.BlockSpec((W2, 2), lambda i: (0, 0)),
            pl.BlockSpec((1, 2), lambda i: (0, 0)),
        ],
        out_specs=pl.BlockSpec((_BLK, 2), lambda i: (i, 0)),
        out_shape=jax.ShapeDtypeStruct((ROWS_G, 2), jnp.float32),
    )(h, agg, deg, Wr2, Wn2, bc2, g2r, be2r, mh, wh2, bh)


def kernel(x, edge_index, W_in, b_in, W_head, b_head,
           Wr0, Wn0, bc0, g0, be0,
           Wr1, Wn1, bc1, g1, be1,
           Wr2, Wn2, bc2, g2, be2):
    xp = x.reshape(P, 2, IN_DIM, N)
    src2d = edge_index[0].reshape(E // CH, CH)
    dst2d = edge_index[1].reshape(E // CH, CH)
    zeros_w2 = jnp.zeros((ZR, W2), jnp.float32)
    ones_w2 = jnp.ones((CH, W2), jnp.float32)
    eye2 = jnp.eye(2, dtype=jnp.float32)
    mh = jnp.kron(eye2, jnp.full((HID, HID), 1.0 / HID, jnp.float32))

    def pack_w(w):
        return jnp.kron(eye2, w)

    def pack_v(v):
        return jnp.tile(v.reshape(1, HID), (1, 2))

    hs = [_tc_input(xp, W_in, pack_v(b_in), k) for k in range(GRP)]
    deg = _get_sc_deg()(dst2d, ones_w2, zeros_w2)[:, :1]
    wh2 = jnp.concatenate(
        [jnp.concatenate([W_head, jnp.zeros((HID, 1), jnp.float32)], 1),
         jnp.concatenate([jnp.zeros((HID, 1), jnp.float32), W_head], 1)],
        axis=0)
    bh2 = jnp.tile(b_head.reshape(1, 1), (1, 2))

    layers = [(Wr0, Wn0, bc0, g0, be0),
              (Wr1, Wn1, bc1, g1, be1),
              (Wr2, Wn2, bc2, g2, be2)]
    for i, (Wr, Wn, bc, g, be) in enumerate(layers):
        aggs = [_get_sc_agg()(hk, src2d, dst2d, zeros_w2) for hk in hs]
        packed = (pack_w(Wr), pack_w(Wn), pack_v(bc), pack_v(g), pack_v(be))
        if i < 2:
            hs = [_tc_layer(hs[k], aggs[k], deg, *packed, mh)
                  for k in range(GRP)]
        else:
            logit2 = jnp.concatenate(
                [_tc_layer_head(hs[k], aggs[k], deg, *packed, mh, wh2, bh2)
                 for k in range(GRP)], axis=0)
    return logit2.reshape(P, N, 2).transpose(0, 2, 1).reshape(B, 64, 64)


# speedup vs baseline: 1.5908x; 1.0003x over previous
import functools

import jax
import jax.numpy as jnp
from jax import lax
from jax.experimental import pallas as pl
from jax.experimental.pallas import tpu as pltpu
from jax.experimental.pallas import tpu_sc as plsc

B = 64
N = 4096
E = 16384
IN_DIM = 12
HID = 64
P = B // 2
W2 = 2 * HID
BN2 = P * N

NC = 2
NS = 16
CH = 128
EPT = E // NS
NCH = EPT // CH
RPT = N // NS
ZR = 64

GRP = 2
PG = P // GRP
ROWS_G = PG * N
PPC = PG // NC


def _sc_agg_body(h_hbm, src_hbm, dst_hbm, zeros_hbm, out_hbm,
                 agg_sh, src_v, adj_v, dst_v, rows_v, zero_v, gsem, ssem,
                 csem):
    cid = lax.axis_index("c")
    sid = lax.axis_index("s")
    pltpu.sync_copy(src_hbm.at[pl.ds(sid * NCH, NCH)], src_v)
    pltpu.sync_copy(dst_hbm.at[pl.ds(sid * NCH, NCH)], dst_v)
    pltpu.sync_copy(zeros_hbm, zero_v)

    my_rows = pl.ds(sid * RPT, RPT)

    def per_pair(g, carry):
        base = (cid * PPC + g) * N
        for r in range(NCH):
            for c in range(CH // 16):
                sl = pl.ds(c * 16, 16)
                adj_v[r, sl] = src_v[r, sl] + base
        @pl.when(g >= 1)
        def _():
            pltpu.make_async_copy(agg_sh.at[my_rows],
                                  out_hbm.at[pl.ds(base, RPT)], csem).wait()
        for z in range(RPT // ZR):
            pltpu.sync_copy(zero_v,
                            agg_sh.at[pl.ds(sid * RPT + z * ZR, ZR)])
        plsc.subcore_barrier()
        pltpu.async_copy(h_hbm.at[adj_v.at[0]], rows_v.at[0], gsem)
        for j in range(NCH):
            b = j % 4
            if j + 1 < NCH:
                if j + 1 >= 4:
                    pltpu.make_async_copy(rows_v.at[(j + 1) % 4],
                                          agg_sh.at[dst_v.at[j]], ssem).wait()
                pltpu.async_copy(h_hbm.at[adj_v.at[j + 1]],
                                 rows_v.at[(j + 1) % 4], gsem)
            pltpu.make_async_copy(h_hbm.at[adj_v.at[j]], rows_v.at[b],
                                  gsem).wait()
            pltpu.async_copy(rows_v.at[b], agg_sh.at[dst_v.at[j]], ssem,
                             add=True)
        for _ in range(4):
            pltpu.make_async_copy(rows_v.at[0], agg_sh.at[dst_v.at[0]],
                                  ssem).wait()
        plsc.subcore_barrier()
        pltpu.async_copy(agg_sh.at[my_rows],
                         out_hbm.at[pl.ds(base + sid * RPT, RPT)], csem)
        return carry

    lax.fori_loop(0, PPC, per_pair, 0)
    pltpu.make_async_copy(agg_sh.at[my_rows],
                          out_hbm.at[pl.ds(0, RPT)], csem).wait()


@functools.cache
def _get_sc_agg():
    return pl.kernel(
        _sc_agg_body,
        out_type=jax.ShapeDtypeStruct((ROWS_G, W2), jnp.float32),
        mesh=plsc.VectorSubcoreMesh(core_axis_name="c", subcore_axis_name="s"),
        scratch_types=[
            pltpu.VMEM_SHARED((N, W2), jnp.float32),
            pltpu.VMEM((NCH, CH), jnp.int32),
            pltpu.VMEM((NCH, CH), jnp.int32),
            pltpu.VMEM((NCH, CH), jnp.int32),
            pltpu.VMEM((4, CH, W2), jnp.float32),
            pltpu.VMEM((ZR, W2), jnp.float32),
            pltpu.SemaphoreType.DMA,
            pltpu.SemaphoreType.DMA,
            pltpu.SemaphoreType.DMA,
        ],
    )


def _sc_deg_body(dst_hbm, ones_hbm, zeros_hbm, out_hbm,
                 deg_sh, dst_v, ones_v, zero_v):
    cid = lax.axis_index("c")
    sid = lax.axis_index("s")

    @pl.when(cid == 0)
    def _():
        pltpu.sync_copy(dst_hbm.at[pl.ds(sid * NCH, NCH)], dst_v)
        pltpu.sync_copy(ones_hbm, ones_v)
        pltpu.sync_copy(zeros_hbm, zero_v)
        for z in range(RPT // ZR):
            pltpu.sync_copy(zero_v,
                            deg_sh.at[pl.ds(sid * RPT + z * ZR, ZR)])
        plsc.subcore_barrier()
        for j in range(NCH):
            pltpu.sync_copy(ones_v, deg_sh.at[dst_v.at[j]], add=True)
        plsc.subcore_barrier()
        pltpu.sync_copy(deg_sh.at[pl.ds(sid * RPT, RPT)],
                        out_hbm.at[pl.ds(sid * RPT, RPT)])


@functools.cache
def _get_sc_deg():
    return pl.kernel(
        _sc_deg_body,
        out_type=jax.ShapeDtypeStruct((N, W2), jnp.float32),
        mesh=plsc.VectorSubcoreMesh(core_axis_name="c", subcore_axis_name="s"),
        scratch_types=[
            pltpu.VMEM_SHARED((N, W2), jnp.float32),
            pltpu.VMEM((NCH, CH), jnp.int32),
            pltpu.VMEM((CH, W2), jnp.float32),
            pltpu.VMEM((ZR, W2), jnp.float32),
        ],
    )


def _tc_in_kernel(x_ref, w_ref, b_ref, out_ref):
    h0 = lax.dot_general(x_ref[0, 0], w_ref[...], (((0,), (0,)), ((), ())),
                         preferred_element_type=jnp.float32)
    h1 = lax.dot_general(x_ref[0, 1], w_ref[...], (((0,), (0,)), ((), ())),
                         preferred_element_type=jnp.float32)
    out_ref[...] = jnp.concatenate([h0, h1], axis=1) + b_ref[...]


def _tc_input(xp, W_in, b2_row, k):
    return pl.pallas_call(
        _tc_in_kernel,
        grid=(PG,),
        in_specs=[
            pl.BlockSpec((1, 2, IN_DIM, N), lambda i: (k * PG + i, 0, 0, 0)),
            pl.BlockSpec((IN_DIM, HID), lambda i: (0, 0)),
            pl.BlockSpec((1, W2), lambda i: (0, 0)),
        ],
        out_specs=pl.BlockSpec((N, W2), lambda i: (i, 0)),
        out_shape=jax.ShapeDtypeStruct((ROWS_G, W2), jnp.float32),
    )(xp, W_in, b2_row)


_BLK = 4096


def _dense_core(h_ref, agg_ref, deg_ref, wr_ref, wn_ref, bc_ref,
                g_ref, be_ref, mh_ref):
    h = h_ref[...]
    a = agg_ref[...] / jnp.maximum(deg_ref[...], 1.0)
    hb = h.astype(jnp.bfloat16)
    ab = a.astype(jnp.bfloat16)
    hn = (lax.dot_general(hb, wr_ref[...].astype(jnp.bfloat16),
                          (((1,), (0,)), ((), ())),
                          preferred_element_type=jnp.float32)
          + lax.dot_general(ab, wn_ref[...].astype(jnp.bfloat16),
                            (((1,), (0,)), ((), ())),
                            preferred_element_type=jnp.float32)
          + bc_ref[...])
    m = lax.dot_general(hn, mh_ref[...], (((1,), (0,)), ((), ())),
                        preferred_element_type=jnp.float32)
    d = hn - m
    v = lax.dot_general(d * d, mh_ref[...], (((1,), (0,)), ((), ())),
                        preferred_element_type=jnp.float32)
    ln = d / jnp.sqrt(v + 1e-5) * g_ref[...] + be_ref[...]
    return h + jnp.maximum(ln, 0.0)


def _tc_layer_kernel(h_ref, agg_ref, deg_ref, wr_ref, wn_ref, bc_ref,
                     g_ref, be_ref, mh_ref, out_ref):
    out_ref[...] = _dense_core(h_ref, agg_ref, deg_ref, wr_ref, wn_ref,
                               bc_ref, g_ref, be_ref, mh_ref)


def _tc_layer_head_kernel(h_ref, agg_ref, deg_ref, wr_ref, wn_ref, bc_ref,
                          g_ref, be_ref, mh_ref, wh_ref, bh_ref, out_ref):
    hf = _dense_core(h_ref, agg_ref, deg_ref, wr_ref, wn_ref,
                     bc_ref, g_ref, be_ref, mh_ref)
    out_ref[...] = (lax.dot_general(hf, wh_ref[...], (((1,), (0,)), ((), ())),
                                    preferred_element_type=jnp.float32)
                    + bh_ref[...])


def _dense_specs():
    return [
        pl.BlockSpec((_BLK, W2), lambda i: (i, 0)),
        pl.BlockSpec((_BLK, W2), lambda i: (i, 0)),
        pl.BlockSpec((N, 1), lambda i: (0, 0)),
        pl.BlockSpec((W2, W2), lambda i: (0, 0)),
        pl.BlockSpec((W2, W2), lambda i: (0, 0)),
        pl.BlockSpec((1, W2), lambda i: (0, 0)),
        pl.BlockSpec((1, W2), lambda i: (0, 0)),
        pl.BlockSpec((1, W2), lambda i: (0, 0)),
        pl.BlockSpec((W2, W2), lambda i: (0, 0)),
    ]


def _tc_layer(h, agg, deg, Wr2, Wn2, bc2, g2r, be2r, mh):
    return pl.pallas_call(
        _tc_layer_kernel,
        grid=(ROWS_G // _BLK,),
        in_specs=_dense_specs(),
        out_specs=pl.BlockSpec((_BLK, W2), lambda i: (i, 0)),
        out_shape=jax.ShapeDtypeStruct((ROWS_G, W2), jnp.float32),
    )(h, agg, deg, Wr2, Wn2, bc2, g2r, be2r, mh)


def _tc_layer_head(h, agg, deg, Wr2, Wn2, bc2, g2r, be2r, mh, wh2, bh):
    return pl.pallas_call(
        _tc_layer_head_kernel,
        grid=(ROWS_G // _BLK,),
        in_specs=_dense_specs() + [
            pl.BlockSpec((W2, 2), lambda i: (0, 0)),
            pl.BlockSpec((1, 2), lambda i: (0, 0)),
        ],
        out_specs=pl.BlockSpec((_BLK, 2), lambda i: (i, 0)),
        out_shape=jax.ShapeDtypeStruct((ROWS_G, 2), jnp.float32),
    )(h, agg, deg, Wr2, Wn2, bc2, g2r, be2r, mh, wh2, bh)


def kernel(x, edge_index, W_in, b_in, W_head, b_head,
           Wr0, Wn0, bc0, g0, be0,
           Wr1, Wn1, bc1, g1, be1,
           Wr2, Wn2, bc2, g2, be2):
    xp = x.reshape(P, 2, IN_DIM, N)
    src2d = edge_index[0].reshape(E // CH, CH)
    dst2d = edge_index[1].reshape(E // CH, CH)
    zeros_w2 = jnp.zeros((ZR, W2), jnp.float32)
    ones_w2 = jnp.ones((CH, W2), jnp.float32)
    eye2 = jnp.eye(2, dtype=jnp.float32)
    mh = jnp.kron(eye2, jnp.full((HID, HID), 1.0 / HID, jnp.float32))

    def pack_w(w):
        return jnp.kron(eye2, w)

    def pack_v(v):
        return jnp.tile(v.reshape(1, HID), (1, 2))

    hs = [_tc_input(xp, W_in, pack_v(b_in), k) for k in range(GRP)]
    deg = _get_sc_deg()(dst2d, ones_w2, zeros_w2)[:, :1]
    wh2 = jnp.concatenate(
        [jnp.concatenate([W_head, jnp.zeros((HID, 1), jnp.float32)], 1),
         jnp.concatenate([jnp.zeros((HID, 1), jnp.float32), W_head], 1)],
        axis=0)
    bh2 = jnp.tile(b_head.reshape(1, 1), (1, 2))

    layers = [(Wr0, Wn0, bc0, g0, be0),
              (Wr1, Wn1, bc1, g1, be1),
              (Wr2, Wn2, bc2, g2, be2)]
    for i, (Wr, Wn, bc, g, be) in enumerate(layers):
        aggs = [_get_sc_agg()(hk, src2d, dst2d, zeros_w2) for hk in hs]
        packed = (pack_w(Wr), pack_w(Wn), pack_v(bc), pack_v(g), pack_v(be))
        if i < 2:
            hs = [_tc_layer(hs[k], aggs[k], deg, *packed, mh)
                  for k in range(GRP)]
        else:
            logit2 = jnp.concatenate(
                [_tc_layer_head(hs[k], aggs[k], deg, *packed, mh, wh2, bh2)
                 for k in range(GRP)], axis=0)
    return logit2.reshape(P, N, 2).transpose(0, 2, 1).reshape(B, 64, 64)
